# Initial kernel scaffold; baseline (speedup 1.0000x reference)
import functools

import jax
import jax.numpy as jnp
from jax import lax
from jax.experimental import pallas as pl
from jax.experimental.pallas import tpu as pltpu
from jax.experimental.pallas import tpu_sc as plsc

_N = 10000
_E = 320000
_D = 128
_NC = 2
_NS = 16
_NW = _NC * _NS
_CHUNK = 128
_CHUNKS = 79
_EPAD = _NW * _CHUNKS * _CHUNK
_NP = 10016
_RPT = _NP // _NS


def _sc_aggregate(x_pad, src_p, dst_p, zeros_np):
    @functools.partial(
        pl.kernel,
        out_type=jax.ShapeDtypeStruct((_NC, _NP, _D), jnp.float32),
        mesh=plsc.VectorSubcoreMesh(core_axis_name="c", subcore_axis_name="s"),
        scratch_types=[
            pltpu.VMEM((_CHUNKS, _CHUNK), jnp.int32),
            pltpu.VMEM((_CHUNKS, 1, _CHUNK), jnp.int32),
            pltpu.VMEM((_CHUNK, _D), jnp.float32),
            pltpu.VMEM_SHARED((_NP, _D), jnp.float32),
            pltpu.SemaphoreType.DMA,
        ],
    )
    def agg_kernel(x_hbm, src_hbm, dst_hbm, z_hbm, out_hbm,
                   src_v, dst_v, rows_v, agg_s, sem):
        c = lax.axis_index("c")
        s = lax.axis_index("s")
        wid = c * _NS + s
        pltpu.sync_copy(src_hbm.at[wid], src_v)
        pltpu.sync_copy(dst_hbm.at[wid], dst_v)
        pltpu.sync_copy(z_hbm.at[pl.ds(s * _RPT, _RPT)],
                        agg_s.at[pl.ds(s * _RPT, _RPT)])
        plsc.subcore_barrier()

        def body(j, carry):
            pltpu.async_copy(x_hbm.at[src_v.at[j]], rows_v, sem).wait()
            pltpu.sync_copy(rows_v, agg_s.at[dst_v.at[j]], add=True)
            return carry

        lax.fori_loop(0, _CHUNKS, body, 0)
        plsc.subcore_barrier()
        pltpu.sync_copy(agg_s.at[pl.ds(s * _RPT, _RPT)],
                        out_hbm.at[c, pl.ds(s * _RPT, _RPT)])

    return agg_kernel(x_pad, src_p, dst_p, zeros_np)


def _tc_linear(partials, x_pad, w_rel, w_root, b):
    def linear_body(p_ref, x_ref, wr_ref, wt_ref, b_ref, o_ref):
        agg = p_ref[0] + p_ref[1]
        o_ref[...] = (
            jnp.dot(agg, wr_ref[...], preferred_element_type=jnp.float32)
            + jnp.dot(x_ref[...], wt_ref[...], preferred_element_type=jnp.float32)
            + b_ref[...]
        )

    return pl.pallas_call(
        linear_body,
        out_shape=jax.ShapeDtypeStruct((_NP, _D), jnp.float32),
    )(partials, x_pad, w_rel, w_root, b.reshape(1, _D))


def kernel(edge_index, features, W1_rel, W1_root, b1, W2_rel, W2_root, b2):
    src = edge_index[0].astype(jnp.int32)
    dst = edge_index[1].astype(jnp.int32)
    pad = _EPAD - _E
    src_p = jnp.concatenate([src, jnp.full((pad,), _N, jnp.int32)])
    src_p = src_p.reshape(_NW, _CHUNKS, _CHUNK)
    dst_p = jnp.concatenate([dst, jnp.full((pad,), _N, jnp.int32)])
    dst_p = dst_p.reshape(_NW, _CHUNKS, 1, _CHUNK)

    x_pad = jnp.zeros((_NP, _D), jnp.float32).at[:_N].set(features)
    zeros_np = jnp.zeros((_NP, _D), jnp.float32)

    p1 = _sc_aggregate(x_pad, src_p, dst_p, zeros_np)
    h_pad = _tc_linear(p1, x_pad, W1_rel, W1_root, b1)

    p2 = _sc_aggregate(h_pad, src_p, dst_p, zeros_np)
    out_pad = _tc_linear(p2, h_pad, W2_rel, W2_root, b2)

    return out_pad[:_N]


# speedup vs baseline: 4.8642x; 4.8642x over previous
import functools

import jax
import jax.numpy as jnp
from jax import lax
from jax.experimental import pallas as pl
from jax.experimental.pallas import tpu as pltpu
from jax.experimental.pallas import tpu_sc as plsc

_N = 10000
_E = 320000
_D = 128
_NC = 2
_NS = 16
_NW = _NC * _NS
_CHUNK = 128
_CHUNKS = 79
_EPAD = _NW * _CHUNKS * _CHUNK
_NP = 10112
_RPT = _NP // _NS


def _sc_aggregate(x_pad, src_p, dst_p, zeros_np):
    @functools.partial(
        pl.kernel,
        out_type=jax.ShapeDtypeStruct((_NC, _NP, _D), jnp.float32),
        mesh=plsc.VectorSubcoreMesh(core_axis_name="c", subcore_axis_name="s"),
        scratch_types=[
            pltpu.VMEM((_CHUNKS, _CHUNK), jnp.int32),
            pltpu.VMEM((_CHUNKS, _CHUNK), jnp.int32),
            pltpu.VMEM((_CHUNK, _D), jnp.float32),
            pltpu.VMEM_SHARED((_NP, _D), jnp.float32),
            pltpu.SemaphoreType.DMA,
        ],
    )
    def agg_kernel(x_hbm, src_hbm, dst_hbm, z_hbm, out_hbm,
                   src_v, dst_v, rows_v, agg_s, sem):
        c = lax.axis_index("c")
        s = lax.axis_index("s")
        wid = c * _NS + s
        pltpu.sync_copy(src_hbm.at[wid], src_v)
        pltpu.sync_copy(dst_hbm.at[wid], dst_v)
        pltpu.sync_copy(z_hbm.at[pl.ds(s * _RPT, _RPT)],
                        agg_s.at[pl.ds(s * _RPT, _RPT)])
        plsc.subcore_barrier()

        def body(j, carry):
            pltpu.async_copy(x_hbm.at[src_v.at[j]], rows_v, sem).wait()
            pltpu.sync_copy(rows_v, agg_s.at[dst_v.at[j]], add=True)
            return carry

        lax.fori_loop(0, _CHUNKS, body, 0)
        plsc.subcore_barrier()
        pltpu.sync_copy(agg_s.at[pl.ds(s * _RPT, _RPT)],
                        out_hbm.at[c, pl.ds(s * _RPT, _RPT)])

    return agg_kernel(x_pad, src_p, dst_p, zeros_np)


def _tc_linear(partials, x_pad, w_rel, w_root, b):
    def linear_body(p_ref, x_ref, wr_ref, wt_ref, b_ref, o_ref):
        agg = p_ref[0] + p_ref[1]
        o_ref[...] = (
            jnp.dot(agg, wr_ref[...], preferred_element_type=jnp.float32)
            + jnp.dot(x_ref[...], wt_ref[...], preferred_element_type=jnp.float32)
            + b_ref[...]
        )

    return pl.pallas_call(
        linear_body,
        out_shape=jax.ShapeDtypeStruct((_NP, _D), jnp.float32),
    )(partials, x_pad, w_rel, w_root, b.reshape(1, _D))


def kernel(edge_index, features, W1_rel, W1_root, b1, W2_rel, W2_root, b2):
    src = edge_index[0].astype(jnp.int32)
    dst = edge_index[1].astype(jnp.int32)
    pad = _EPAD - _E
    src_p = jnp.concatenate([src, jnp.full((pad,), _N, jnp.int32)])
    src_p = src_p.reshape(_NW, _CHUNKS, _CHUNK)
    dst_p = jnp.concatenate([dst, jnp.full((pad,), _N, jnp.int32)])
    dst_p = dst_p.reshape(_NW, _CHUNKS, _CHUNK)

    x_pad = jnp.zeros((_NP, _D), jnp.float32).at[:_N].set(features)
    zeros_np = jnp.zeros((_NP, _D), jnp.float32)

    p1 = _sc_aggregate(x_pad, src_p, dst_p, zeros_np)
    h_pad = _tc_linear(p1, x_pad, W1_rel, W1_root, b1)

    p2 = _sc_aggregate(h_pad, src_p, dst_p, zeros_np)
    out_pad = _tc_linear(p2, h_pad, W2_rel, W2_root, b2)

    return out_pad[:_N]
